# baseline (speedup 1.0000x reference)
import jax
import jax.numpy as jnp
from jax import lax
from jax.experimental import pallas as pl
from jax.experimental.pallas import tpu as pltpu
from jax.experimental.pallas import tpu_sc as plsc

BATCH = 8
N_ROWS = 10000
ROWS_PER_TILE = 632
PAD_ROWS = ROWS_PER_TILE * 16
LAST_TILE_ROWS = N_ROWS - 15 * ROWS_PER_TILE
E_EDGES = 160000
CHUNK = 128
VCHUNK = CHUNK * 16
N_WORKERS = 32
EDGES_PER_WORKER = E_EDGES // N_WORKERS
NFULL = EDGES_PER_WORKER // CHUNK
TAIL = EDGES_PER_WORKER - NFULL * CHUNK
NPAIRS = NFULL // 2
GROUPS = CHUNK // 16
OUT_CHUNK_ROWS = 40
N_OUT_CHUNKS = N_ROWS // OUT_CHUNK_ROWS
OUT_ITERS = (N_OUT_CHUNKS + N_WORKERS - 1) // N_WORKERS


def _stage_x(xstage_v, xt_v, iota, nrows, b):
    ridx = iota >> 2
    cidx = (iota & 3) * 8 + b

    @plsc.parallel_loop(0, (nrows * 4) // 16, unroll=4)
    def tbody(t):
        v = xstage_v[pl.ds(t * 16, 16)]
        plsc.store_scatter(xt_v, [ridx + 4 * t, cidx], v)


XP_ROWS = 320
XP_LAST = N_ROWS - 31 * XP_ROWS


def _xpose_body(x, x4out, xstage_v, xt_v):
    cid = lax.axis_index("c")
    sid = lax.axis_index("s")
    wid = sid * 2 + cid
    iota = lax.iota(jnp.int32, 16)

    def dostage(nrows):
        r0 = wid * XP_ROWS
        for b in range(BATCH):
            pltpu.sync_copy(x.at[pl.ds(b * 4 * N_ROWS + 4 * r0, 4 * nrows)],
                            xstage_v.at[pl.ds(0, 4 * nrows)])
            _stage_x(xstage_v, xt_v, iota, nrows, b)
        pltpu.sync_copy(xt_v.at[pl.ds(0, nrows)], x4out.at[pl.ds(r0, nrows)])

    @pl.when(wid < 31)
    def _():
        dostage(XP_ROWS)

    @pl.when(wid == 31)
    def _():
        dostage(XP_LAST)


def _spmm_body(x4, eps_w, wm, wlv, indices, zeros_rows, partial,
               vals_v,
               eps_a, wm_a, wlv_a, idx0_a, idx1_a, col_a, row_a, xg_a, og_a,
               eps_b, wm_b, wlv_b, idx0_b, idx1_b, col_b, row_b, xg_b, og_b,
               col_t8, row_t8,
               acc,
               sin_a, sin_b, sg_a, sg_b, ss_a, ss_b):
    cid = lax.axis_index("c")
    sid = lax.axis_index("s")
    wid = sid * 2 + cid
    iota = lax.iota(jnp.int32, 16)

    pltpu.sync_copy(zeros_rows,
                    acc.at[pl.ds(sid * ROWS_PER_TILE, ROWS_PER_TILE)])
    plsc.subcore_barrier()

    wbase = wid * EDGES_PER_WORKER

    def issue_inputs(bufs, sem, ebase):
        e_v, m_v, lv_v, i0_v, i1_v = bufs
        vbase = ebase * 16
        pltpu.async_copy(eps_w.at[pl.ds(vbase, VCHUNK)], e_v, sem)
        pltpu.async_copy(wm.at[pl.ds(vbase, VCHUNK)], m_v, sem)
        pltpu.async_copy(wlv.at[pl.ds(vbase, VCHUNK)], lv_v, sem)
        pltpu.async_copy(indices.at[pl.ds(vbase, VCHUNK)], i0_v, sem)
        pltpu.async_copy(indices.at[pl.ds(E_EDGES * 16 + vbase, VCHUNK)],
                         i1_v, sem)

    def wait_inputs(bufs, sem):
        e_v, m_v, lv_v, i0_v, i1_v = bufs
        pltpu.make_async_copy(eps_w.at[pl.ds(0, VCHUNK)], e_v, sem).wait()
        pltpu.make_async_copy(wm.at[pl.ds(0, VCHUNK)], m_v, sem).wait()
        pltpu.make_async_copy(wlv.at[pl.ds(0, VCHUNK)], lv_v, sem).wait()
        pltpu.make_async_copy(indices.at[pl.ds(0, VCHUNK)], i0_v,
                              sem).wait()
        pltpu.make_async_copy(indices.at[pl.ds(0, VCHUNK)], i1_v,
                              sem).wait()

    def extract(src_v, dst_v, k):
        g = iota * 16 + k * 256
        dst_v[pl.ds(k * 16, 16)] = plsc.load_gather(src_v, [g]) >> 2

    def compute_vals(e_v, lv_v, m_v, n16):
        tidx = iota * CHUNK

        @plsc.parallel_loop(0, n16, unroll=8)
        def vbody(k):
            o = k * 16
            v = (e_v[pl.ds(o, 16)] * jnp.exp(lv_v[pl.ds(o, 16)])
                 + m_v[pl.ds(o, 16)])
            plsc.store_scatter(vals_v, [tidx + k], v)

    def block_products(xg_v, og_v, groups):
        for g in range(groups):
            erow = iota + g * 16
            vv = [[vals_v[pl.ds((4 * i + j) * CHUNK + g * 16, 16)]
                   for j in range(4)] for i in range(4)]
            for b in range(BATCH):
                xx = [plsc.load_gather(
                          xg_v,
                          [erow, jnp.full((16,), 8 * i + b, jnp.int32)])
                      for i in range(4)]
                for j in range(4):
                    o = vv[0][j] * xx[0]
                    o = o + vv[1][j] * xx[1]
                    o = o + vv[2][j] * xx[2]
                    o = o + vv[3][j] * xx[3]
                    plsc.store_scatter(
                        og_v,
                        [erow, jnp.full((16,), 8 * j + b, jnp.int32)], o)

    def process(bufs, col_v, row_v, xg_v, og_v, sg, ss, first):
        e_v, m_v, lv_v, i0_v, i1_v = bufs
        for k in range(CHUNK // 16):
            extract(i1_v, col_v, k)
        pltpu.async_copy(x4.at[col_v], xg_v, sg)
        @pl.when(jnp.logical_not(first))
        def _():
            pltpu.make_async_copy(og_v, acc.at[row_v], ss).wait()
        for k in range(CHUNK // 16):
            extract(i0_v, row_v, k)
        compute_vals(e_v, lv_v, m_v, VCHUNK // 16)
        pltpu.make_async_copy(x4.at[col_v], xg_v, sg).wait()
        block_products(xg_v, og_v, GROUPS)
        pltpu.async_copy(og_v, acc.at[row_v], ss, add=True)

    bufs_a = (eps_a, wm_a, wlv_a, idx0_a, idx1_a)
    bufs_b = (eps_b, wm_b, wlv_b, idx0_b, idx1_b)

    issue_inputs(bufs_a, sin_a, wbase)

    def pair_body(t, carry):
        ea = wbase + t * (2 * CHUNK)
        wait_inputs(bufs_a, sin_a)
        issue_inputs(bufs_b, sin_b, ea + CHUNK)
        process(bufs_a, col_a, row_a, xg_a, og_a, sg_a, ss_a, t == 0)
        wait_inputs(bufs_b, sin_b)
        issue_inputs(bufs_a, sin_a, ea + 2 * CHUNK)
        process(bufs_b, col_b, row_b, xg_b, og_b, sg_b, ss_b, t == 0)
        return carry

    lax.fori_loop(0, NPAIRS, pair_body, 0)

    wait_inputs(bufs_a, sin_a)
    process(bufs_a, col_a, row_a, xg_a, og_a, sg_a, ss_a, NPAIRS == 0)

    tbase = wbase + NFULL * CHUNK
    tv = tbase * 16
    cps = [
        pltpu.async_copy(eps_w.at[pl.ds(tv, TAIL * 16)],
                         eps_b.at[pl.ds(0, TAIL * 16)], sin_b),
        pltpu.async_copy(wlv.at[pl.ds(tv, TAIL * 16)],
                         wlv_b.at[pl.ds(0, TAIL * 16)], sin_b),
        pltpu.async_copy(wm.at[pl.ds(tv, TAIL * 16)],
                         wm_b.at[pl.ds(0, TAIL * 16)], sin_b),
        pltpu.async_copy(indices.at[pl.ds(tv, TAIL * 16)],
                         idx0_b.at[pl.ds(0, TAIL * 16)], sin_b),
        pltpu.async_copy(indices.at[pl.ds(E_EDGES * 16 + tv, TAIL * 16)],
                         idx1_b.at[pl.ds(0, TAIL * 16)], sin_b),
    ]
    pltpu.make_async_copy(og_b, acc.at[row_b], ss_b).wait()
    for cp in cps:
        cp.wait()
    tmask = iota < TAIL
    g0 = iota * 16
    colt = jnp.where(tmask, plsc.load_gather(idx1_b, [g0]) >> 2, 0)
    rowt = jnp.where(tmask, plsc.load_gather(idx0_b, [g0]) >> 2, 0)
    plsc.store_scatter(col_t8, [iota], colt, mask=tmask)
    plsc.store_scatter(row_t8, [iota], rowt, mask=tmask)
    pltpu.async_copy(x4.at[col_t8], xg_b.at[pl.ds(0, TAIL)], sg_b).wait()
    compute_vals(eps_b, wlv_b, wm_b, TAIL)
    block_products(xg_b, og_b, 1)
    pltpu.async_copy(og_b.at[pl.ds(0, TAIL)], acc.at[row_t8], ss_b,
                     add=True).wait()
    pltpu.make_async_copy(og_a, acc.at[row_a], ss_a).wait()

    plsc.subcore_barrier()
    pltpu.sync_copy(
        acc.at[pl.ds(sid * ROWS_PER_TILE, ROWS_PER_TILE)],
        partial.at[pl.ds(cid * PAD_ROWS + sid * ROWS_PER_TILE,
                         ROWS_PER_TILE)])


def _combine_body(partial, bm, blv, eps_b, out,
                  p0_v, p1_v, bm_v, blv_v, epsb_v, bias_v, out_v, outT_v,
                  sem):
    cid = lax.axis_index("c")
    sid = lax.axis_index("s")
    wid = sid * 2 + cid
    iota = lax.iota(jnp.int32, 16)
    half = iota >> 3

    def body(t, carry):
        c = t * N_WORKERS + wid

        @pl.when(c < N_OUT_CHUNKS)
        def _():
            r0 = c * OUT_CHUNK_ROWS
            ob = c * (4 * OUT_CHUNK_ROWS)
            nb = 4 * OUT_CHUNK_ROWS
            cps = [
                pltpu.async_copy(partial.at[pl.ds(r0, OUT_CHUNK_ROWS)],
                                 p0_v, sem),
                pltpu.async_copy(partial.at[pl.ds(PAD_ROWS + r0,
                                                  OUT_CHUNK_ROWS)],
                                 p1_v, sem),
                pltpu.async_copy(bm.at[pl.ds(ob, nb)], bm_v, sem),
                pltpu.async_copy(blv.at[pl.ds(ob, nb)], blv_v, sem),
                pltpu.async_copy(eps_b.at[pl.ds(ob, nb)], epsb_v, sem),
            ]
            for cp in cps:
                cp.wait()
            for h in range(nb // 16):
                s = pl.ds(h * 16, 16)
                bias_v[s] = epsb_v[s] * jnp.exp(blv_v[s]) + bm_v[s]
            for r in range(OUT_CHUNK_ROWS):
                for h in range(2):
                    bp = plsc.load_gather(bias_v, [half + (4 * r + 2 * h)])
                    s = pl.ds(h * 16, 16)
                    out_v[r, s] = p0_v[r, s] + p1_v[r, s] + bp
            for b in range(BATCH):
                for h in range(nb // 16):
                    g = plsc.load_gather(
                        out_v, [(iota + 16 * h) >> 2, (iota & 3) * 8 + b])
                    outT_v[b, pl.ds(h * 16, 16)] = g
            wcps = [pltpu.async_copy(outT_v.at[b],
                                     out.at[b, pl.ds(ob, nb)], sem)
                    for b in range(BATCH)]
            for cp in wcps:
                cp.wait()
        return carry

    lax.fori_loop(0, OUT_ITERS, body, 0)


def kernel(x, weight_mean, weight_log_var, b_mean, b_log_var, eps_w, eps_b,
           indices):
    xf = x.reshape(BATCH * 4 * N_ROWS)
    idxf = indices.reshape(2 * E_EDGES * 16)
    zeros_rows = jnp.zeros((ROWS_PER_TILE, 32), jnp.float32)

    mesh = plsc.VectorSubcoreMesh(core_axis_name="c", subcore_axis_name="s")
    cparams = pltpu.CompilerParams(needs_layout_passes=False,
                                   use_tc_tiling_on_sc=False)
    vbuf = lambda: pltpu.VMEM((VCHUNK,), jnp.float32)
    ibuf = lambda: pltpu.VMEM((VCHUNK,), jnp.int32)
    ebuf = lambda: pltpu.VMEM((CHUNK,), jnp.int32)
    gbuf = lambda: pltpu.VMEM((CHUNK, 32), jnp.float32)
    xpose = pl.kernel(
        _xpose_body,
        compiler_params=cparams,
        out_type=jax.ShapeDtypeStruct((N_ROWS, 32), jnp.float32),
        mesh=mesh,
        scratch_types=[
            pltpu.VMEM((XP_ROWS * 4,), jnp.float32),
            pltpu.VMEM((XP_ROWS, 32), jnp.float32),
        ],
    )
    x4 = xpose(xf)
    spmm = pl.kernel(
        _spmm_body,
        compiler_params=cparams,
        out_type=jax.ShapeDtypeStruct((2 * PAD_ROWS, 32), jnp.float32),
        mesh=mesh,
        scratch_types=[
            vbuf(),
            vbuf(), vbuf(), vbuf(), ibuf(), ibuf(),
            ebuf(), ebuf(), gbuf(), gbuf(),
            vbuf(), vbuf(), vbuf(), ibuf(), ibuf(),
            ebuf(), ebuf(), gbuf(), gbuf(),
            pltpu.VMEM((TAIL,), jnp.int32),
            pltpu.VMEM((TAIL,), jnp.int32),
            pltpu.VMEM_SHARED((PAD_ROWS, 32), jnp.float32),
            pltpu.SemaphoreType.DMA, pltpu.SemaphoreType.DMA,
            pltpu.SemaphoreType.DMA, pltpu.SemaphoreType.DMA,
            pltpu.SemaphoreType.DMA, pltpu.SemaphoreType.DMA,
        ],
    )
    partial = spmm(x4, eps_w, weight_mean, weight_log_var, idxf,
                   zeros_rows)

    combine = pl.kernel(
        _combine_body,
        compiler_params=cparams,
        out_type=jax.ShapeDtypeStruct((BATCH, 4 * N_ROWS), jnp.float32),
        mesh=mesh,
        scratch_types=[
            pltpu.VMEM((OUT_CHUNK_ROWS, 32), jnp.float32),
            pltpu.VMEM((OUT_CHUNK_ROWS, 32), jnp.float32),
            pltpu.VMEM((4 * OUT_CHUNK_ROWS,), jnp.float32),
            pltpu.VMEM((4 * OUT_CHUNK_ROWS,), jnp.float32),
            pltpu.VMEM((4 * OUT_CHUNK_ROWS,), jnp.float32),
            pltpu.VMEM((4 * OUT_CHUNK_ROWS,), jnp.float32),
            pltpu.VMEM((OUT_CHUNK_ROWS, 32), jnp.float32),
            pltpu.VMEM((BATCH, 4 * OUT_CHUNK_ROWS), jnp.float32),
            pltpu.SemaphoreType.DMA,
        ],
    )
    out = combine(partial, b_mean, b_log_var, eps_b)

    return (out.reshape(BATCH, 4 * N_ROWS, 1), jnp.asarray(0.0, jnp.float32))


# speedup vs baseline: 1.0013x; 1.0013x over previous
import jax
import jax.numpy as jnp
from jax import lax
from jax.experimental import pallas as pl
from jax.experimental.pallas import tpu as pltpu
from jax.experimental.pallas import tpu_sc as plsc

BATCH = 8
N_ROWS = 10000
ROWS_PER_TILE = 632
PAD_ROWS = ROWS_PER_TILE * 16
LAST_TILE_ROWS = N_ROWS - 15 * ROWS_PER_TILE
E_EDGES = 160000
CHUNK = 128
VCHUNK = CHUNK * 16
N_WORKERS = 32
EDGES_PER_WORKER = E_EDGES // N_WORKERS
NFULL = EDGES_PER_WORKER // CHUNK
TAIL = EDGES_PER_WORKER - NFULL * CHUNK
NPAIRS = NFULL // 2
GROUPS = CHUNK // 16
OUT_CHUNK_ROWS = 40
N_OUT_CHUNKS = N_ROWS // OUT_CHUNK_ROWS
OUT_ITERS = (N_OUT_CHUNKS + N_WORKERS - 1) // N_WORKERS


def _stage_x(xstage_v, xt_v, iota, nrows, b):
    ridx = iota >> 2
    cidx = (iota & 3) * 8 + b

    @plsc.parallel_loop(0, (nrows * 4) // 16, unroll=4)
    def tbody(t):
        v = xstage_v[pl.ds(t * 16, 16)]
        plsc.store_scatter(xt_v, [ridx + 4 * t, cidx], v)


XP_ROWS = 320
XP_LAST = N_ROWS - 31 * XP_ROWS


def _xpose_body(x, x4out, xstage_v, xt_v):
    cid = lax.axis_index("c")
    sid = lax.axis_index("s")
    wid = sid * 2 + cid
    iota = lax.iota(jnp.int32, 16)

    def dostage(nrows):
        r0 = wid * XP_ROWS
        for b in range(BATCH):
            pltpu.sync_copy(x.at[pl.ds(b * 4 * N_ROWS + 4 * r0, 4 * nrows)],
                            xstage_v.at[pl.ds(0, 4 * nrows)])
            _stage_x(xstage_v, xt_v, iota, nrows, b)
        pltpu.sync_copy(xt_v.at[pl.ds(0, nrows)], x4out.at[pl.ds(r0, nrows)])

    @pl.when(wid < 31)
    def _():
        dostage(XP_ROWS)

    @pl.when(wid == 31)
    def _():
        dostage(XP_LAST)


def _spmm_body(x4, eps_w, wm, wlv, indices, zeros_rows, partial,
               vals_v,
               eps_a, wm_a, wlv_a, idx0_a, idx1_a,
               ca0, ca1, ca2, ca3, ra0, ra1, ra2, ra3, xg_a, og_a,
               eps_b, wm_b, wlv_b, idx0_b, idx1_b,
               cb0, cb1, cb2, cb3, rb0, rb1, rb2, rb3, xg_b, og_b,
               col_t8, row_t8,
               acc,
               sin_a, sin_b, sg_a, sg_b, ss_a, ss_b):
    cid = lax.axis_index("c")
    sid = lax.axis_index("s")
    wid = sid * 2 + cid
    iota = lax.iota(jnp.int32, 16)

    pltpu.sync_copy(zeros_rows,
                    acc.at[pl.ds(sid * ROWS_PER_TILE, ROWS_PER_TILE)])
    plsc.subcore_barrier()

    wbase = wid * EDGES_PER_WORKER

    def issue_inputs(bufs, sem, ebase):
        e_v, m_v, lv_v, i0_v, i1_v = bufs
        vbase = ebase * 16
        pltpu.async_copy(eps_w.at[pl.ds(vbase, VCHUNK)], e_v, sem)
        pltpu.async_copy(wm.at[pl.ds(vbase, VCHUNK)], m_v, sem)
        pltpu.async_copy(wlv.at[pl.ds(vbase, VCHUNK)], lv_v, sem)
        pltpu.async_copy(indices.at[pl.ds(vbase, VCHUNK)], i0_v, sem)
        pltpu.async_copy(indices.at[pl.ds(E_EDGES * 16 + vbase, VCHUNK)],
                         i1_v, sem)

    def wait_inputs(bufs, sem):
        e_v, m_v, lv_v, i0_v, i1_v = bufs
        pltpu.make_async_copy(eps_w.at[pl.ds(0, VCHUNK)], e_v, sem).wait()
        pltpu.make_async_copy(wm.at[pl.ds(0, VCHUNK)], m_v, sem).wait()
        pltpu.make_async_copy(wlv.at[pl.ds(0, VCHUNK)], lv_v, sem).wait()
        pltpu.make_async_copy(indices.at[pl.ds(0, VCHUNK)], i0_v,
                              sem).wait()
        pltpu.make_async_copy(indices.at[pl.ds(0, VCHUNK)], i1_v,
                              sem).wait()

    def extract(src_v, dsts, k):
        g = iota * 16 + k * 256
        dsts[k // 2][pl.ds((k % 2) * 16, 16)] = (
            plsc.load_gather(src_v, [g]) >> 2)

    def compute_vals(e_v, lv_v, m_v, n16):
        tidx = iota * CHUNK

        @plsc.parallel_loop(0, n16, unroll=8)
        def vbody(k):
            o = k * 16
            v = (e_v[pl.ds(o, 16)] * jnp.exp(lv_v[pl.ds(o, 16)])
                 + m_v[pl.ds(o, 16)])
            plsc.store_scatter(vals_v, [tidx + k], v)

    def block_products(xg_v, og_v, groups):
        for g in range(groups):
            erow = iota + g * 16
            vv = [[vals_v[pl.ds((4 * i + j) * CHUNK + g * 16, 16)]
                   for j in range(4)] for i in range(4)]
            for b in range(BATCH):
                xx = [plsc.load_gather(
                          xg_v,
                          [erow, jnp.full((16,), 8 * i + b, jnp.int32)])
                      for i in range(4)]
                for j in range(4):
                    o = vv[0][j] * xx[0]
                    o = o + vv[1][j] * xx[1]
                    o = o + vv[2][j] * xx[2]
                    o = o + vv[3][j] * xx[3]
                    plsc.store_scatter(
                        og_v,
                        [erow, jnp.full((16,), 8 * j + b, jnp.int32)], o)

    def process(bufs, col_v, row_v, xg_v, og_v, sg, ss, first):
        e_v, m_v, lv_v, i0_v, i1_v = bufs
        for k in range(CHUNK // 16):
            extract(i1_v, col_v, k)
        for s in range(4):
            pltpu.async_copy(x4.at[col_v[s]],
                             xg_v.at[pl.ds(32 * s, 32)], sg)
        @pl.when(jnp.logical_not(first))
        def _():
            for s in range(4):
                pltpu.make_async_copy(og_v.at[pl.ds(32 * s, 32)],
                                      acc.at[row_v[s]], ss).wait()
        for k in range(CHUNK // 16):
            extract(i0_v, row_v, k)
        compute_vals(e_v, lv_v, m_v, VCHUNK // 16)
        for s in range(4):
            pltpu.make_async_copy(x4.at[col_v[s]],
                                  xg_v.at[pl.ds(32 * s, 32)], sg).wait()
        block_products(xg_v, og_v, GROUPS)
        for s in range(4):
            pltpu.async_copy(og_v.at[pl.ds(32 * s, 32)], acc.at[row_v[s]],
                             ss, add=True)

    bufs_a = (eps_a, wm_a, wlv_a, idx0_a, idx1_a)
    bufs_b = (eps_b, wm_b, wlv_b, idx0_b, idx1_b)
    col_a = (ca0, ca1, ca2, ca3)
    row_a = (ra0, ra1, ra2, ra3)
    col_b = (cb0, cb1, cb2, cb3)
    row_b = (rb0, rb1, rb2, rb3)

    issue_inputs(bufs_a, sin_a, wbase)

    def pair_body(t, carry):
        ea = wbase + t * (2 * CHUNK)
        wait_inputs(bufs_a, sin_a)
        issue_inputs(bufs_b, sin_b, ea + CHUNK)
        process(bufs_a, col_a, row_a, xg_a, og_a, sg_a, ss_a, t == 0)
        wait_inputs(bufs_b, sin_b)
        issue_inputs(bufs_a, sin_a, ea + 2 * CHUNK)
        process(bufs_b, col_b, row_b, xg_b, og_b, sg_b, ss_b, t == 0)
        return carry

    lax.fori_loop(0, NPAIRS, pair_body, 0)

    wait_inputs(bufs_a, sin_a)
    process(bufs_a, col_a, row_a, xg_a, og_a, sg_a, ss_a, NPAIRS == 0)

    tbase = wbase + NFULL * CHUNK
    tv = tbase * 16
    cps = [
        pltpu.async_copy(eps_w.at[pl.ds(tv, TAIL * 16)],
                         eps_b.at[pl.ds(0, TAIL * 16)], sin_b),
        pltpu.async_copy(wlv.at[pl.ds(tv, TAIL * 16)],
                         wlv_b.at[pl.ds(0, TAIL * 16)], sin_b),
        pltpu.async_copy(wm.at[pl.ds(tv, TAIL * 16)],
                         wm_b.at[pl.ds(0, TAIL * 16)], sin_b),
        pltpu.async_copy(indices.at[pl.ds(tv, TAIL * 16)],
                         idx0_b.at[pl.ds(0, TAIL * 16)], sin_b),
        pltpu.async_copy(indices.at[pl.ds(E_EDGES * 16 + tv, TAIL * 16)],
                         idx1_b.at[pl.ds(0, TAIL * 16)], sin_b),
    ]
    for s in range(4):
        pltpu.make_async_copy(og_b.at[pl.ds(32 * s, 32)],
                              acc.at[row_b[s]], ss_b).wait()
    for cp in cps:
        cp.wait()
    tmask = iota < TAIL
    g0 = iota * 16
    colt = jnp.where(tmask, plsc.load_gather(idx1_b, [g0]) >> 2, 0)
    rowt = jnp.where(tmask, plsc.load_gather(idx0_b, [g0]) >> 2, 0)
    plsc.store_scatter(col_t8, [iota], colt, mask=tmask)
    plsc.store_scatter(row_t8, [iota], rowt, mask=tmask)
    pltpu.async_copy(x4.at[col_t8], xg_b.at[pl.ds(0, TAIL)], sg_b).wait()
    compute_vals(eps_b, wlv_b, wm_b, TAIL)
    block_products(xg_b, og_b, 1)
    pltpu.async_copy(og_b.at[pl.ds(0, TAIL)], acc.at[row_t8], ss_b,
                     add=True).wait()
    for s in range(4):
        pltpu.make_async_copy(og_a.at[pl.ds(32 * s, 32)],
                              acc.at[row_a[s]], ss_a).wait()

    plsc.subcore_barrier()
    pltpu.sync_copy(
        acc.at[pl.ds(sid * ROWS_PER_TILE, ROWS_PER_TILE)],
        partial.at[pl.ds(cid * PAD_ROWS + sid * ROWS_PER_TILE,
                         ROWS_PER_TILE)])


def _combine_body(partial, bm, blv, eps_b, out,
                  p0_v, p1_v, bm_v, blv_v, epsb_v, bias_v, out_v, outT_v,
                  sem):
    cid = lax.axis_index("c")
    sid = lax.axis_index("s")
    wid = sid * 2 + cid
    iota = lax.iota(jnp.int32, 16)
    half = iota >> 3

    def body(t, carry):
        c = t * N_WORKERS + wid

        @pl.when(c < N_OUT_CHUNKS)
        def _():
            r0 = c * OUT_CHUNK_ROWS
            ob = c * (4 * OUT_CHUNK_ROWS)
            nb = 4 * OUT_CHUNK_ROWS
            cps = [
                pltpu.async_copy(partial.at[pl.ds(r0, OUT_CHUNK_ROWS)],
                                 p0_v, sem),
                pltpu.async_copy(partial.at[pl.ds(PAD_ROWS + r0,
                                                  OUT_CHUNK_ROWS)],
                                 p1_v, sem),
                pltpu.async_copy(bm.at[pl.ds(ob, nb)], bm_v, sem),
                pltpu.async_copy(blv.at[pl.ds(ob, nb)], blv_v, sem),
                pltpu.async_copy(eps_b.at[pl.ds(ob, nb)], epsb_v, sem),
            ]
            for cp in cps:
                cp.wait()
            for h in range(nb // 16):
                s = pl.ds(h * 16, 16)
                bias_v[s] = epsb_v[s] * jnp.exp(blv_v[s]) + bm_v[s]
            for r in range(OUT_CHUNK_ROWS):
                for h in range(2):
                    bp = plsc.load_gather(bias_v, [half + (4 * r + 2 * h)])
                    s = pl.ds(h * 16, 16)
                    out_v[r, s] = p0_v[r, s] + p1_v[r, s] + bp
            for b in range(BATCH):
                for h in range(nb // 16):
                    g = plsc.load_gather(
                        out_v, [(iota + 16 * h) >> 2, (iota & 3) * 8 + b])
                    outT_v[b, pl.ds(h * 16, 16)] = g
            wcps = [pltpu.async_copy(outT_v.at[b],
                                     out.at[b, pl.ds(ob, nb)], sem)
                    for b in range(BATCH)]
            for cp in wcps:
                cp.wait()
        return carry

    lax.fori_loop(0, OUT_ITERS, body, 0)


def kernel(x, weight_mean, weight_log_var, b_mean, b_log_var, eps_w, eps_b,
           indices):
    xf = x.reshape(BATCH * 4 * N_ROWS)
    idxf = indices.reshape(2 * E_EDGES * 16)
    zeros_rows = jnp.zeros((ROWS_PER_TILE, 32), jnp.float32)

    mesh = plsc.VectorSubcoreMesh(core_axis_name="c", subcore_axis_name="s")
    cparams = pltpu.CompilerParams(needs_layout_passes=False,
                                   use_tc_tiling_on_sc=False)
    vbuf = lambda: pltpu.VMEM((VCHUNK,), jnp.float32)
    ibuf = lambda: pltpu.VMEM((VCHUNK,), jnp.int32)
    ebuf = lambda: pltpu.VMEM((CHUNK // 4,), jnp.int32)
    gbuf = lambda: pltpu.VMEM((CHUNK, 32), jnp.float32)
    xpose = pl.kernel(
        _xpose_body,
        compiler_params=cparams,
        out_type=jax.ShapeDtypeStruct((N_ROWS, 32), jnp.float32),
        mesh=mesh,
        scratch_types=[
            pltpu.VMEM((XP_ROWS * 4,), jnp.float32),
            pltpu.VMEM((XP_ROWS, 32), jnp.float32),
        ],
    )
    x4 = xpose(xf)
    spmm = pl.kernel(
        _spmm_body,
        compiler_params=cparams,
        out_type=jax.ShapeDtypeStruct((2 * PAD_ROWS, 32), jnp.float32),
        mesh=mesh,
        scratch_types=[
            vbuf(),
            vbuf(), vbuf(), vbuf(), ibuf(), ibuf(),
            ebuf(), ebuf(), ebuf(), ebuf(),
            ebuf(), ebuf(), ebuf(), ebuf(),
            gbuf(), gbuf(),
            vbuf(), vbuf(), vbuf(), ibuf(), ibuf(),
            ebuf(), ebuf(), ebuf(), ebuf(),
            ebuf(), ebuf(), ebuf(), ebuf(),
            gbuf(), gbuf(),
            pltpu.VMEM((TAIL,), jnp.int32),
            pltpu.VMEM((TAIL,), jnp.int32),
            pltpu.VMEM_SHARED((PAD_ROWS, 32), jnp.float32),
            pltpu.SemaphoreType.DMA, pltpu.SemaphoreType.DMA,
            pltpu.SemaphoreType.DMA, pltpu.SemaphoreType.DMA,
            pltpu.SemaphoreType.DMA, pltpu.SemaphoreType.DMA,
        ],
    )
    partial = spmm(x4, eps_w, weight_mean, weight_log_var, idxf,
                   zeros_rows)

    combine = pl.kernel(
        _combine_body,
        compiler_params=cparams,
        out_type=jax.ShapeDtypeStruct((BATCH, 4 * N_ROWS), jnp.float32),
        mesh=mesh,
        scratch_types=[
            pltpu.VMEM((OUT_CHUNK_ROWS, 32), jnp.float32),
            pltpu.VMEM((OUT_CHUNK_ROWS, 32), jnp.float32),
            pltpu.VMEM((4 * OUT_CHUNK_ROWS,), jnp.float32),
            pltpu.VMEM((4 * OUT_CHUNK_ROWS,), jnp.float32),
            pltpu.VMEM((4 * OUT_CHUNK_ROWS,), jnp.float32),
            pltpu.VMEM((4 * OUT_CHUNK_ROWS,), jnp.float32),
            pltpu.VMEM((OUT_CHUNK_ROWS, 32), jnp.float32),
            pltpu.VMEM((BATCH, 4 * OUT_CHUNK_ROWS), jnp.float32),
            pltpu.SemaphoreType.DMA,
        ],
    )
    out = combine(partial, b_mean, b_log_var, eps_b)

    return (out.reshape(BATCH, 4 * N_ROWS, 1), jnp.asarray(0.0, jnp.float32))


# speedup vs baseline: 1.2711x; 1.2694x over previous
import jax
import jax.numpy as jnp
from jax import lax
from jax.experimental import pallas as pl
from jax.experimental.pallas import tpu as pltpu
from jax.experimental.pallas import tpu_sc as plsc

BATCH = 8
N_ROWS = 10000
ROWS_PER_TILE = 632
PAD_ROWS = ROWS_PER_TILE * 16
LAST_TILE_ROWS = N_ROWS - 15 * ROWS_PER_TILE
E_EDGES = 160000
CHUNK = 128
VCHUNK = CHUNK * 16
N_WORKERS = 32
EDGES_PER_WORKER = E_EDGES // N_WORKERS
NFULL = EDGES_PER_WORKER // CHUNK
TAIL = EDGES_PER_WORKER - NFULL * CHUNK
NPAIRS = NFULL // 2
GROUPS = CHUNK // 16
OUT_CHUNK_ROWS = 40
N_OUT_CHUNKS = N_ROWS // OUT_CHUNK_ROWS
OUT_ITERS = (N_OUT_CHUNKS + N_WORKERS - 1) // N_WORKERS


def _stage_x(xstage_v, xt_v, iota, nrows, b):
    ridx = iota >> 2
    cidx = (iota & 3) * 8 + b

    @plsc.parallel_loop(0, (nrows * 4) // 16, unroll=4)
    def tbody(t):
        v = xstage_v[pl.ds(t * 16, 16)]
        plsc.store_scatter(xt_v, [ridx + 4 * t, cidx], v)


XP_ROWS = 320
XP_LAST = N_ROWS - 31 * XP_ROWS


def _xpose_body(x, x4out, xstage_v, xt_v):
    cid = lax.axis_index("c")
    sid = lax.axis_index("s")
    wid = sid * 2 + cid
    iota = lax.iota(jnp.int32, 16)

    def dostage(nrows):
        r0 = wid * XP_ROWS
        for b in range(BATCH):
            pltpu.sync_copy(x.at[pl.ds(b * 4 * N_ROWS + 4 * r0, 4 * nrows)],
                            xstage_v.at[pl.ds(0, 4 * nrows)])
            _stage_x(xstage_v, xt_v, iota, nrows, b)
        pltpu.sync_copy(xt_v.at[pl.ds(0, nrows)], x4out.at[pl.ds(r0, nrows)])

    @pl.when(wid < 31)
    def _():
        dostage(XP_ROWS)

    @pl.when(wid == 31)
    def _():
        dostage(XP_LAST)


def _spmm_body(x4, eps_w, wm, wlv, indices, zeros_rows, partial,
               vals_v, xgs_v, ogs_v,
               eps_a, wm_a, wlv_a, idx0_a, idx1_a,
               ca0, ca1, ca2, ca3, ra0, ra1, ra2, ra3, xg_a, og_a,
               eps_b, wm_b, wlv_b, idx0_b, idx1_b,
               cb0, cb1, cb2, cb3, rb0, rb1, rb2, rb3, xg_b, og_b,
               col_t8, row_t8,
               acc,
               sin_a, sin_b, sg_a, sg_b, ss_a, ss_b):
    cid = lax.axis_index("c")
    sid = lax.axis_index("s")
    wid = sid * 2 + cid
    iota = lax.iota(jnp.int32, 16)

    pltpu.sync_copy(zeros_rows,
                    acc.at[pl.ds(sid * ROWS_PER_TILE, ROWS_PER_TILE)])
    plsc.subcore_barrier()

    wbase = wid * EDGES_PER_WORKER

    def issue_inputs(bufs, sem, ebase):
        e_v, m_v, lv_v, i0_v, i1_v = bufs
        vbase = ebase * 16
        pltpu.async_copy(eps_w.at[pl.ds(vbase, VCHUNK)], e_v, sem)
        pltpu.async_copy(wm.at[pl.ds(vbase, VCHUNK)], m_v, sem)
        pltpu.async_copy(wlv.at[pl.ds(vbase, VCHUNK)], lv_v, sem)
        pltpu.async_copy(indices.at[pl.ds(vbase, VCHUNK)], i0_v, sem)
        pltpu.async_copy(indices.at[pl.ds(E_EDGES * 16 + vbase, VCHUNK)],
                         i1_v, sem)

    def wait_inputs(bufs, sem):
        e_v, m_v, lv_v, i0_v, i1_v = bufs
        pltpu.make_async_copy(eps_w.at[pl.ds(0, VCHUNK)], e_v, sem).wait()
        pltpu.make_async_copy(wm.at[pl.ds(0, VCHUNK)], m_v, sem).wait()
        pltpu.make_async_copy(wlv.at[pl.ds(0, VCHUNK)], lv_v, sem).wait()
        pltpu.make_async_copy(indices.at[pl.ds(0, VCHUNK)], i0_v,
                              sem).wait()
        pltpu.make_async_copy(indices.at[pl.ds(0, VCHUNK)], i1_v,
                              sem).wait()

    def extract(src_v, dsts, k):
        g = iota * 16 + k * 256
        dsts[k // 2][pl.ds((k % 2) * 16, 16)] = (
            plsc.load_gather(src_v, [g]) >> 2)

    def compute_vals(e_v, lv_v, m_v, n16):
        tidx = iota * 129

        @plsc.parallel_loop(0, n16, unroll=8)
        def vbody(k):
            o = k * 16
            v = (e_v[pl.ds(o, 16)] * jnp.exp(lv_v[pl.ds(o, 16)])
                 + m_v[pl.ds(o, 16)])
            plsc.store_scatter(vals_v, [tidx + k], v)

    def skew_in(xg_v, n2):
        @plsc.parallel_loop(0, n2, unroll=8)
        def cb(w):
            e = w >> 1
            h = (w & 1) * 16
            xgs_v[pl.ds(e * 33 + h, 16)] = xg_v[e, pl.ds(h, 16)]

    def skew_out(og_v, n2):
        @plsc.parallel_loop(0, n2, unroll=8)
        def cb(w):
            e = w >> 1
            h = (w & 1) * 16
            og_v[e, pl.ds(h, 16)] = ogs_v[pl.ds(e * 33 + h, 16)]

    def block_products(groups):
        for g in range(groups):
            vv = [[vals_v[pl.ds((4 * i + j) * 129 + g * 16, 16)]
                   for j in range(4)] for i in range(4)]
            ebase = iota * 33 + g * (16 * 33)
            for b in range(BATCH):
                xx = [plsc.load_gather(xgs_v, [ebase + (8 * i + b)])
                      for i in range(4)]
                for j in range(4):
                    o = vv[0][j] * xx[0]
                    o = o + vv[1][j] * xx[1]
                    o = o + vv[2][j] * xx[2]
                    o = o + vv[3][j] * xx[3]
                    plsc.store_scatter(ogs_v, [ebase + (8 * j + b)], o)

    def process(bufs, col_v, row_v, xg_v, og_v, sg, ss, first):
        e_v, m_v, lv_v, i0_v, i1_v = bufs
        for k in range(CHUNK // 16):
            extract(i1_v, col_v, k)
        for s in range(4):
            pltpu.async_copy(x4.at[col_v[s]],
                             xg_v.at[pl.ds(32 * s, 32)], sg)
        @pl.when(jnp.logical_not(first))
        def _():
            for s in range(4):
                pltpu.make_async_copy(og_v.at[pl.ds(32 * s, 32)],
                                      acc.at[row_v[s]], ss).wait()
        for k in range(CHUNK // 16):
            extract(i0_v, row_v, k)
        compute_vals(e_v, lv_v, m_v, VCHUNK // 16)
        for s in range(4):
            pltpu.make_async_copy(x4.at[col_v[s]],
                                  xg_v.at[pl.ds(32 * s, 32)], sg).wait()
        skew_in(xg_v, 2 * CHUNK)
        block_products(GROUPS)
        skew_out(og_v, 2 * CHUNK)
        for s in range(4):
            pltpu.async_copy(og_v.at[pl.ds(32 * s, 32)], acc.at[row_v[s]],
                             ss, add=True)

    bufs_a = (eps_a, wm_a, wlv_a, idx0_a, idx1_a)
    bufs_b = (eps_b, wm_b, wlv_b, idx0_b, idx1_b)
    col_a = (ca0, ca1, ca2, ca3)
    row_a = (ra0, ra1, ra2, ra3)
    col_b = (cb0, cb1, cb2, cb3)
    row_b = (rb0, rb1, rb2, rb3)

    issue_inputs(bufs_a, sin_a, wbase)

    def pair_body(t, carry):
        ea = wbase + t * (2 * CHUNK)
        wait_inputs(bufs_a, sin_a)
        issue_inputs(bufs_b, sin_b, ea + CHUNK)
        process(bufs_a, col_a, row_a, xg_a, og_a, sg_a, ss_a, t == 0)
        wait_inputs(bufs_b, sin_b)
        issue_inputs(bufs_a, sin_a, ea + 2 * CHUNK)
        process(bufs_b, col_b, row_b, xg_b, og_b, sg_b, ss_b, t == 0)
        return carry

    lax.fori_loop(0, NPAIRS, pair_body, 0)

    wait_inputs(bufs_a, sin_a)
    process(bufs_a, col_a, row_a, xg_a, og_a, sg_a, ss_a, NPAIRS == 0)

    tbase = wbase + NFULL * CHUNK
    tv = tbase * 16
    cps = [
        pltpu.async_copy(eps_w.at[pl.ds(tv, TAIL * 16)],
                         eps_b.at[pl.ds(0, TAIL * 16)], sin_b),
        pltpu.async_copy(wlv.at[pl.ds(tv, TAIL * 16)],
                         wlv_b.at[pl.ds(0, TAIL * 16)], sin_b),
        pltpu.async_copy(wm.at[pl.ds(tv, TAIL * 16)],
                         wm_b.at[pl.ds(0, TAIL * 16)], sin_b),
        pltpu.async_copy(indices.at[pl.ds(tv, TAIL * 16)],
                         idx0_b.at[pl.ds(0, TAIL * 16)], sin_b),
        pltpu.async_copy(indices.at[pl.ds(E_EDGES * 16 + tv, TAIL * 16)],
                         idx1_b.at[pl.ds(0, TAIL * 16)], sin_b),
    ]
    for s in range(4):
        pltpu.make_async_copy(og_b.at[pl.ds(32 * s, 32)],
                              acc.at[row_b[s]], ss_b).wait()
    for cp in cps:
        cp.wait()
    tmask = iota < TAIL
    g0 = iota * 16
    colt = jnp.where(tmask, plsc.load_gather(idx1_b, [g0]) >> 2, 0)
    rowt = jnp.where(tmask, plsc.load_gather(idx0_b, [g0]) >> 2, 0)
    plsc.store_scatter(col_t8, [iota], colt, mask=tmask)
    plsc.store_scatter(row_t8, [iota], rowt, mask=tmask)
    pltpu.async_copy(x4.at[col_t8], xg_b.at[pl.ds(0, TAIL)], sg_b).wait()
    compute_vals(eps_b, wlv_b, wm_b, TAIL)
    skew_in(xg_b, 2 * TAIL)
    block_products(1)
    skew_out(og_b, 2 * TAIL)
    pltpu.async_copy(og_b.at[pl.ds(0, TAIL)], acc.at[row_t8], ss_b,
                     add=True).wait()
    for s in range(4):
        pltpu.make_async_copy(og_a.at[pl.ds(32 * s, 32)],
                              acc.at[row_a[s]], ss_a).wait()

    plsc.subcore_barrier()
    pltpu.sync_copy(
        acc.at[pl.ds(sid * ROWS_PER_TILE, ROWS_PER_TILE)],
        partial.at[pl.ds(cid * PAD_ROWS + sid * ROWS_PER_TILE,
                         ROWS_PER_TILE)])


def _combine_body(partial, bm, blv, eps_b, out,
                  p0_v, p1_v, bm_v, blv_v, epsb_v, bias_v, out_v, outT_v,
                  sem):
    cid = lax.axis_index("c")
    sid = lax.axis_index("s")
    wid = sid * 2 + cid
    iota = lax.iota(jnp.int32, 16)
    half = iota >> 3

    def body(t, carry):
        c = t * N_WORKERS + wid

        @pl.when(c < N_OUT_CHUNKS)
        def _():
            r0 = c * OUT_CHUNK_ROWS
            ob = c * (4 * OUT_CHUNK_ROWS)
            nb = 4 * OUT_CHUNK_ROWS
            cps = [
                pltpu.async_copy(partial.at[pl.ds(r0, OUT_CHUNK_ROWS)],
                                 p0_v, sem),
                pltpu.async_copy(partial.at[pl.ds(PAD_ROWS + r0,
                                                  OUT_CHUNK_ROWS)],
                                 p1_v, sem),
                pltpu.async_copy(bm.at[pl.ds(ob, nb)], bm_v, sem),
                pltpu.async_copy(blv.at[pl.ds(ob, nb)], blv_v, sem),
                pltpu.async_copy(eps_b.at[pl.ds(ob, nb)], epsb_v, sem),
            ]
            for cp in cps:
                cp.wait()
            for h in range(nb // 16):
                s = pl.ds(h * 16, 16)
                bias_v[s] = epsb_v[s] * jnp.exp(blv_v[s]) + bm_v[s]
            for r in range(OUT_CHUNK_ROWS):
                for h in range(2):
                    bp = plsc.load_gather(bias_v, [half + (4 * r + 2 * h)])
                    s = pl.ds(h * 16, 16)
                    out_v[r, s] = p0_v[r, s] + p1_v[r, s] + bp
            for b in range(BATCH):
                for h in range(nb // 16):
                    g = plsc.load_gather(
                        out_v, [(iota + 16 * h) >> 2, (iota & 3) * 8 + b])
                    outT_v[b, pl.ds(h * 16, 16)] = g
            wcps = [pltpu.async_copy(outT_v.at[b],
                                     out.at[b, pl.ds(ob, nb)], sem)
                    for b in range(BATCH)]
            for cp in wcps:
                cp.wait()
        return carry

    lax.fori_loop(0, OUT_ITERS, body, 0)


def kernel(x, weight_mean, weight_log_var, b_mean, b_log_var, eps_w, eps_b,
           indices):
    xf = x.reshape(BATCH * 4 * N_ROWS)
    idxf = indices.reshape(2 * E_EDGES * 16)
    zeros_rows = jnp.zeros((ROWS_PER_TILE, 32), jnp.float32)

    mesh = plsc.VectorSubcoreMesh(core_axis_name="c", subcore_axis_name="s")
    cparams = pltpu.CompilerParams(needs_layout_passes=False,
                                   use_tc_tiling_on_sc=False)
    vbuf = lambda: pltpu.VMEM((VCHUNK,), jnp.float32)
    ibuf = lambda: pltpu.VMEM((VCHUNK,), jnp.int32)
    ebuf = lambda: pltpu.VMEM((CHUNK // 4,), jnp.int32)
    gbuf = lambda: pltpu.VMEM((CHUNK, 32), jnp.float32)
    xpose = pl.kernel(
        _xpose_body,
        compiler_params=cparams,
        out_type=jax.ShapeDtypeStruct((N_ROWS, 32), jnp.float32),
        mesh=mesh,
        scratch_types=[
            pltpu.VMEM((XP_ROWS * 4,), jnp.float32),
            pltpu.VMEM((XP_ROWS, 32), jnp.float32),
        ],
    )
    x4 = xpose(xf)
    spmm = pl.kernel(
        _spmm_body,
        compiler_params=cparams,
        out_type=jax.ShapeDtypeStruct((2 * PAD_ROWS, 32), jnp.float32),
        mesh=mesh,
        scratch_types=[
            pltpu.VMEM((16 * 129,), jnp.float32),
            pltpu.VMEM((CHUNK * 33,), jnp.float32),
            pltpu.VMEM((CHUNK * 33,), jnp.float32),
            vbuf(), vbuf(), vbuf(), ibuf(), ibuf(),
            ebuf(), ebuf(), ebuf(), ebuf(),
            ebuf(), ebuf(), ebuf(), ebuf(),
            gbuf(), gbuf(),
            vbuf(), vbuf(), vbuf(), ibuf(), ibuf(),
            ebuf(), ebuf(), ebuf(), ebuf(),
            ebuf(), ebuf(), ebuf(), ebuf(),
            gbuf(), gbuf(),
            pltpu.VMEM((TAIL,), jnp.int32),
            pltpu.VMEM((TAIL,), jnp.int32),
            pltpu.VMEM_SHARED((PAD_ROWS, 32), jnp.float32),
            pltpu.SemaphoreType.DMA, pltpu.SemaphoreType.DMA,
            pltpu.SemaphoreType.DMA, pltpu.SemaphoreType.DMA,
            pltpu.SemaphoreType.DMA, pltpu.SemaphoreType.DMA,
        ],
    )
    partial = spmm(x4, eps_w, weight_mean, weight_log_var, idxf,
                   zeros_rows)

    combine = pl.kernel(
        _combine_body,
        compiler_params=cparams,
        out_type=jax.ShapeDtypeStruct((BATCH, 4 * N_ROWS), jnp.float32),
        mesh=mesh,
        scratch_types=[
            pltpu.VMEM((OUT_CHUNK_ROWS, 32), jnp.float32),
            pltpu.VMEM((OUT_CHUNK_ROWS, 32), jnp.float32),
            pltpu.VMEM((4 * OUT_CHUNK_ROWS,), jnp.float32),
            pltpu.VMEM((4 * OUT_CHUNK_ROWS,), jnp.float32),
            pltpu.VMEM((4 * OUT_CHUNK_ROWS,), jnp.float32),
            pltpu.VMEM((4 * OUT_CHUNK_ROWS,), jnp.float32),
            pltpu.VMEM((OUT_CHUNK_ROWS, 32), jnp.float32),
            pltpu.VMEM((BATCH, 4 * OUT_CHUNK_ROWS), jnp.float32),
            pltpu.SemaphoreType.DMA,
        ],
    )
    out = combine(partial, b_mean, b_log_var, eps_b)

    return (out.reshape(BATCH, 4 * N_ROWS, 1), jnp.asarray(0.0, jnp.float32))


# speedup vs baseline: 1.3029x; 1.0250x over previous
import jax
import jax.numpy as jnp
from jax import lax
from jax.experimental import pallas as pl
from jax.experimental.pallas import tpu as pltpu
from jax.experimental.pallas import tpu_sc as plsc

BATCH = 8
N_ROWS = 10000
ROWS_PER_TILE = 632
PAD_ROWS = ROWS_PER_TILE * 16
LAST_TILE_ROWS = N_ROWS - 15 * ROWS_PER_TILE
E_EDGES = 160000
CHUNK = 128
VCHUNK = CHUNK * 16
N_WORKERS = 32
EDGES_PER_WORKER = E_EDGES // N_WORKERS
NFULL = EDGES_PER_WORKER // CHUNK
TAIL = EDGES_PER_WORKER - NFULL * CHUNK
NPAIRS = NFULL // 2
GROUPS = CHUNK // 16
OUT_CHUNK_ROWS = 40
N_OUT_CHUNKS = N_ROWS // OUT_CHUNK_ROWS
OUT_ITERS = (N_OUT_CHUNKS + N_WORKERS - 1) // N_WORKERS


def _stage_x(xstage_v, xt_v, iota, nrows, b):
    ridx = iota >> 2
    cidx = (iota & 3) * 8 + b

    @plsc.parallel_loop(0, (nrows * 4) // 16, unroll=4)
    def tbody(t):
        v = xstage_v[pl.ds(t * 16, 16)]
        plsc.store_scatter(xt_v, [ridx + 4 * t, cidx], v)


XP_ROWS = 320
XP_LAST = N_ROWS - 31 * XP_ROWS


def _xpose_body(x, x4out, xstage_v, xt_v):
    cid = lax.axis_index("c")
    sid = lax.axis_index("s")
    wid = sid * 2 + cid
    iota = lax.iota(jnp.int32, 16)

    def dostage(nrows):
        r0 = wid * XP_ROWS
        for b in range(BATCH):
            pltpu.sync_copy(x.at[pl.ds(b * 4 * N_ROWS + 4 * r0, 4 * nrows)],
                            xstage_v.at[pl.ds(0, 4 * nrows)])
            _stage_x(xstage_v, xt_v, iota, nrows, b)
        pltpu.sync_copy(xt_v.at[pl.ds(0, nrows)], x4out.at[pl.ds(r0, nrows)])

    @pl.when(wid < 31)
    def _():
        dostage(XP_ROWS)

    @pl.when(wid == 31)
    def _():
        dostage(XP_LAST)


def _spmm_body(x4, eps_w, wm, wlv, indices, zeros_rows, partial,
               vals_v, xgs_v, ogs_v,
               eps_a, wm_a, wlv_a, idx0_a, idx1_a,
               ca0, ca1, ca2, ca3, ra0, ra1, ra2, ra3, xg_a, og_a,
               eps_b, wm_b, wlv_b, idx0_b, idx1_b,
               cb0, cb1, cb2, cb3, rb0, rb1, rb2, rb3, xg_b, og_b,
               col_t8, row_t8,
               acc,
               sin_a, sin_b, sg_a, sg_b, ss_a, ss_b):
    cid = lax.axis_index("c")
    sid = lax.axis_index("s")
    wid = sid * 2 + cid
    iota = lax.iota(jnp.int32, 16)

    pltpu.sync_copy(zeros_rows,
                    acc.at[pl.ds(sid * ROWS_PER_TILE, ROWS_PER_TILE)])
    plsc.subcore_barrier()

    wbase = wid * EDGES_PER_WORKER

    def issue_inputs(bufs, sem, ebase):
        e_v, m_v, lv_v, i0_v, i1_v = bufs
        vbase = ebase * 16
        pltpu.async_copy(eps_w.at[pl.ds(vbase, VCHUNK)], e_v, sem)
        pltpu.async_copy(wm.at[pl.ds(vbase, VCHUNK)], m_v, sem)
        pltpu.async_copy(wlv.at[pl.ds(vbase, VCHUNK)], lv_v, sem)
        pltpu.async_copy(indices.at[pl.ds(vbase, VCHUNK)], i0_v, sem)
        pltpu.async_copy(indices.at[pl.ds(E_EDGES * 16 + vbase, VCHUNK)],
                         i1_v, sem)

    def wait_inputs(bufs, sem):
        e_v, m_v, lv_v, i0_v, i1_v = bufs
        pltpu.make_async_copy(eps_w.at[pl.ds(0, VCHUNK)], e_v, sem).wait()
        pltpu.make_async_copy(wm.at[pl.ds(0, VCHUNK)], m_v, sem).wait()
        pltpu.make_async_copy(wlv.at[pl.ds(0, VCHUNK)], lv_v, sem).wait()
        pltpu.make_async_copy(indices.at[pl.ds(0, VCHUNK)], i0_v,
                              sem).wait()
        pltpu.make_async_copy(indices.at[pl.ds(0, VCHUNK)], i1_v,
                              sem).wait()

    def extract(src_v, dsts, k):
        g = iota * 16 + k * 256
        dsts[k // 2][pl.ds((k % 2) * 16, 16)] = (
            plsc.load_gather(src_v, [g]) >> 2)

    def compute_vals(e_v, lv_v, m_v, n16):
        tidx = iota * 129

        @plsc.parallel_loop(0, n16, unroll=8)
        def vbody(k):
            o = k * 16
            v = (e_v[pl.ds(o, 16)] * jnp.exp(lv_v[pl.ds(o, 16)])
                 + m_v[pl.ds(o, 16)])
            plsc.store_scatter(vals_v, [tidx + k], v)

    def skew_in(xg_v, n2):
        @plsc.parallel_loop(0, n2, unroll=8)
        def cb(w):
            e = w >> 1
            h = (w & 1) * 16
            xgs_v[pl.ds(e * 33 + h, 16)] = xg_v[e, pl.ds(h, 16)]

    def skew_out(og_v, n2):
        @plsc.parallel_loop(0, n2, unroll=8)
        def cb(w):
            e = w >> 1
            h = (w & 1) * 16
            og_v[e, pl.ds(h, 16)] = ogs_v[pl.ds(e * 33 + h, 16)]

    def block_products(groups):
        for g in range(groups):
            vv = [[vals_v[pl.ds((4 * i + j) * 129 + g * 16, 16)]
                   for j in range(4)] for i in range(4)]
            ebase = iota * 33 + g * (16 * 33)
            for b in range(BATCH):
                xx = [plsc.load_gather(xgs_v, [ebase + (8 * i + b)])
                      for i in range(4)]
                for j in range(4):
                    o = vv[0][j] * xx[0]
                    o = o + vv[1][j] * xx[1]
                    o = o + vv[2][j] * xx[2]
                    o = o + vv[3][j] * xx[3]
                    plsc.store_scatter(ogs_v, [ebase + (8 * j + b)], o)

    def stage_gather(bufs, col_v, xg_v, sg):
        i1_v = bufs[4]
        for k in range(CHUNK // 16):
            extract(i1_v, col_v, k)
        for s in range(4):
            pltpu.async_copy(x4.at[col_v[s]],
                             xg_v.at[pl.ds(32 * s, 32)], sg)

    def stage_compute(bufs, col_v, row_v, xg_v, og_v, sg, ss, first):
        e_v, m_v, lv_v, i0_v, i1_v = bufs
        @pl.when(jnp.logical_not(first))
        def _():
            for s in range(4):
                pltpu.make_async_copy(og_v.at[pl.ds(32 * s, 32)],
                                      acc.at[row_v[s]], ss).wait()
        for k in range(CHUNK // 16):
            extract(i0_v, row_v, k)
        compute_vals(e_v, lv_v, m_v, VCHUNK // 16)
        for s in range(4):
            pltpu.make_async_copy(x4.at[col_v[s]],
                                  xg_v.at[pl.ds(32 * s, 32)], sg).wait()
        skew_in(xg_v, 2 * CHUNK)
        block_products(GROUPS)
        skew_out(og_v, 2 * CHUNK)
        for s in range(4):
            pltpu.async_copy(og_v.at[pl.ds(32 * s, 32)], acc.at[row_v[s]],
                             ss, add=True)

    bufs_a = (eps_a, wm_a, wlv_a, idx0_a, idx1_a)
    bufs_b = (eps_b, wm_b, wlv_b, idx0_b, idx1_b)
    col_a = (ca0, ca1, ca2, ca3)
    row_a = (ra0, ra1, ra2, ra3)
    col_b = (cb0, cb1, cb2, cb3)
    row_b = (rb0, rb1, rb2, rb3)

    issue_inputs(bufs_a, sin_a, wbase)
    issue_inputs(bufs_b, sin_b, wbase + CHUNK)

    def pair_body(t, carry):
        ea = wbase + t * (2 * CHUNK)
        wait_inputs(bufs_a, sin_a)
        stage_gather(bufs_a, col_a, xg_a, sg_a)
        wait_inputs(bufs_b, sin_b)
        stage_gather(bufs_b, col_b, xg_b, sg_b)
        stage_compute(bufs_a, col_a, row_a, xg_a, og_a, sg_a, ss_a, t == 0)
        issue_inputs(bufs_a, sin_a, ea + 2 * CHUNK)
        stage_compute(bufs_b, col_b, row_b, xg_b, og_b, sg_b, ss_b, t == 0)

        @pl.when(t < NPAIRS - 1)
        def _():
            issue_inputs(bufs_b, sin_b, ea + 3 * CHUNK)
        return carry

    lax.fori_loop(0, NPAIRS, pair_body, 0)

    wait_inputs(bufs_a, sin_a)
    stage_gather(bufs_a, col_a, xg_a, sg_a)
    stage_compute(bufs_a, col_a, row_a, xg_a, og_a, sg_a, ss_a,
                  NPAIRS == 0)

    tbase = wbase + NFULL * CHUNK
    tv = tbase * 16
    cps = [
        pltpu.async_copy(eps_w.at[pl.ds(tv, TAIL * 16)],
                         eps_b.at[pl.ds(0, TAIL * 16)], sin_b),
        pltpu.async_copy(wlv.at[pl.ds(tv, TAIL * 16)],
                         wlv_b.at[pl.ds(0, TAIL * 16)], sin_b),
        pltpu.async_copy(wm.at[pl.ds(tv, TAIL * 16)],
                         wm_b.at[pl.ds(0, TAIL * 16)], sin_b),
        pltpu.async_copy(indices.at[pl.ds(tv, TAIL * 16)],
                         idx0_b.at[pl.ds(0, TAIL * 16)], sin_b),
        pltpu.async_copy(indices.at[pl.ds(E_EDGES * 16 + tv, TAIL * 16)],
                         idx1_b.at[pl.ds(0, TAIL * 16)], sin_b),
    ]
    for s in range(4):
        pltpu.make_async_copy(og_b.at[pl.ds(32 * s, 32)],
                              acc.at[row_b[s]], ss_b).wait()
    for cp in cps:
        cp.wait()
    tmask = iota < TAIL
    g0 = iota * 16
    colt = jnp.where(tmask, plsc.load_gather(idx1_b, [g0]) >> 2, 0)
    rowt = jnp.where(tmask, plsc.load_gather(idx0_b, [g0]) >> 2, 0)
    plsc.store_scatter(col_t8, [iota], colt, mask=tmask)
    plsc.store_scatter(row_t8, [iota], rowt, mask=tmask)
    pltpu.async_copy(x4.at[col_t8], xg_b.at[pl.ds(0, TAIL)], sg_b).wait()
    compute_vals(eps_b, wlv_b, wm_b, TAIL)
    skew_in(xg_b, 2 * TAIL)
    block_products(1)
    skew_out(og_b, 2 * TAIL)
    pltpu.async_copy(og_b.at[pl.ds(0, TAIL)], acc.at[row_t8], ss_b,
                     add=True).wait()
    for s in range(4):
        pltpu.make_async_copy(og_a.at[pl.ds(32 * s, 32)],
                              acc.at[row_a[s]], ss_a).wait()

    plsc.subcore_barrier()
    pltpu.sync_copy(
        acc.at[pl.ds(sid * ROWS_PER_TILE, ROWS_PER_TILE)],
        partial.at[pl.ds(cid * PAD_ROWS + sid * ROWS_PER_TILE,
                         ROWS_PER_TILE)])


def _combine_body(partial, bm, blv, eps_b, out,
                  p0_v, p1_v, bm_v, blv_v, epsb_v, bias_v, out_v, outT_v,
                  sem):
    cid = lax.axis_index("c")
    sid = lax.axis_index("s")
    wid = sid * 2 + cid
    iota = lax.iota(jnp.int32, 16)
    half = iota >> 3

    def body(t, carry):
        c = t * N_WORKERS + wid

        @pl.when(c < N_OUT_CHUNKS)
        def _():
            r0 = c * OUT_CHUNK_ROWS
            ob = c * (4 * OUT_CHUNK_ROWS)
            nb = 4 * OUT_CHUNK_ROWS
            cps = [
                pltpu.async_copy(partial.at[pl.ds(r0, OUT_CHUNK_ROWS)],
                                 p0_v, sem),
                pltpu.async_copy(partial.at[pl.ds(PAD_ROWS + r0,
                                                  OUT_CHUNK_ROWS)],
                                 p1_v, sem),
                pltpu.async_copy(bm.at[pl.ds(ob, nb)], bm_v, sem),
                pltpu.async_copy(blv.at[pl.ds(ob, nb)], blv_v, sem),
                pltpu.async_copy(eps_b.at[pl.ds(ob, nb)], epsb_v, sem),
            ]
            for cp in cps:
                cp.wait()
            for h in range(nb // 16):
                s = pl.ds(h * 16, 16)
                bias_v[s] = epsb_v[s] * jnp.exp(blv_v[s]) + bm_v[s]
            for r in range(OUT_CHUNK_ROWS):
                for h in range(2):
                    bp = plsc.load_gather(bias_v, [half + (4 * r + 2 * h)])
                    s = pl.ds(h * 16, 16)
                    out_v[r, s] = p0_v[r, s] + p1_v[r, s] + bp
            for b in range(BATCH):
                for h in range(nb // 16):
                    g = plsc.load_gather(
                        out_v, [(iota + 16 * h) >> 2, (iota & 3) * 8 + b])
                    outT_v[b, pl.ds(h * 16, 16)] = g
            wcps = [pltpu.async_copy(outT_v.at[b],
                                     out.at[b, pl.ds(ob, nb)], sem)
                    for b in range(BATCH)]
            for cp in wcps:
                cp.wait()
        return carry

    lax.fori_loop(0, OUT_ITERS, body, 0)


def kernel(x, weight_mean, weight_log_var, b_mean, b_log_var, eps_w, eps_b,
           indices):
    xf = x.reshape(BATCH * 4 * N_ROWS)
    idxf = indices.reshape(2 * E_EDGES * 16)
    zeros_rows = jnp.zeros((ROWS_PER_TILE, 32), jnp.float32)

    mesh = plsc.VectorSubcoreMesh(core_axis_name="c", subcore_axis_name="s")
    cparams = pltpu.CompilerParams(needs_layout_passes=False,
                                   use_tc_tiling_on_sc=False)
    vbuf = lambda: pltpu.VMEM((VCHUNK,), jnp.float32)
    ibuf = lambda: pltpu.VMEM((VCHUNK,), jnp.int32)
    ebuf = lambda: pltpu.VMEM((CHUNK // 4,), jnp.int32)
    gbuf = lambda: pltpu.VMEM((CHUNK, 32), jnp.float32)
    xpose = pl.kernel(
        _xpose_body,
        compiler_params=cparams,
        out_type=jax.ShapeDtypeStruct((N_ROWS, 32), jnp.float32),
        mesh=mesh,
        scratch_types=[
            pltpu.VMEM((XP_ROWS * 4,), jnp.float32),
            pltpu.VMEM((XP_ROWS, 32), jnp.float32),
        ],
    )
    x4 = xpose(xf)
    spmm = pl.kernel(
        _spmm_body,
        compiler_params=cparams,
        out_type=jax.ShapeDtypeStruct((2 * PAD_ROWS, 32), jnp.float32),
        mesh=mesh,
        scratch_types=[
            pltpu.VMEM((16 * 129,), jnp.float32),
            pltpu.VMEM((CHUNK * 33,), jnp.float32),
            pltpu.VMEM((CHUNK * 33,), jnp.float32),
            vbuf(), vbuf(), vbuf(), ibuf(), ibuf(),
            ebuf(), ebuf(), ebuf(), ebuf(),
            ebuf(), ebuf(), ebuf(), ebuf(),
            gbuf(), gbuf(),
            vbuf(), vbuf(), vbuf(), ibuf(), ibuf(),
            ebuf(), ebuf(), ebuf(), ebuf(),
            ebuf(), ebuf(), ebuf(), ebuf(),
            gbuf(), gbuf(),
            pltpu.VMEM((TAIL,), jnp.int32),
            pltpu.VMEM((TAIL,), jnp.int32),
            pltpu.VMEM_SHARED((PAD_ROWS, 32), jnp.float32),
            pltpu.SemaphoreType.DMA, pltpu.SemaphoreType.DMA,
            pltpu.SemaphoreType.DMA, pltpu.SemaphoreType.DMA,
            pltpu.SemaphoreType.DMA, pltpu.SemaphoreType.DMA,
        ],
    )
    partial = spmm(x4, eps_w, weight_mean, weight_log_var, idxf,
                   zeros_rows)

    combine = pl.kernel(
        _combine_body,
        compiler_params=cparams,
        out_type=jax.ShapeDtypeStruct((BATCH, 4 * N_ROWS), jnp.float32),
        mesh=mesh,
        scratch_types=[
            pltpu.VMEM((OUT_CHUNK_ROWS, 32), jnp.float32),
            pltpu.VMEM((OUT_CHUNK_ROWS, 32), jnp.float32),
            pltpu.VMEM((4 * OUT_CHUNK_ROWS,), jnp.float32),
            pltpu.VMEM((4 * OUT_CHUNK_ROWS,), jnp.float32),
            pltpu.VMEM((4 * OUT_CHUNK_ROWS,), jnp.float32),
            pltpu.VMEM((4 * OUT_CHUNK_ROWS,), jnp.float32),
            pltpu.VMEM((OUT_CHUNK_ROWS, 32), jnp.float32),
            pltpu.VMEM((BATCH, 4 * OUT_CHUNK_ROWS), jnp.float32),
            pltpu.SemaphoreType.DMA,
        ],
    )
    out = combine(partial, b_mean, b_log_var, eps_b)

    return (out.reshape(BATCH, 4 * N_ROWS, 1), jnp.asarray(0.0, jnp.float32))
